# baseline (speedup 1.0000x reference)
import jax
import jax.numpy as jnp
from jax import lax
from jax.experimental import pallas as pl
from jax.experimental.pallas import tpu as pltpu
from jax.experimental.pallas import tpu_sc as plsc

_B = 2048
_E = 16
_CK = 64
_TB = 256


def _shift_load(sbuf, off):
    return sbuf[pl.ds(off, 16)]


def _dispatch_body(eidx_hbm, x_hbm, xs_hbm, pos_hbm, ends_hbm, gridx_hbm,
                   eidx_v, sbuf, stage_v, grid_v, cnt_ref, start_ref, xv,
                   pos_v, endv, sem):
    c = lax.axis_index("c")
    s = lax.axis_index("s")
    iota = lax.iota(jnp.int32, 16)
    zero_v = iota * 0
    one_v = zero_v + 1
    sent_v = zero_v - 1

    pltpu.sync_copy(eidx_hbm.at[pl.ds(s * 128, 128)], eidx_v)
    sbuf[pl.ds(0, 16)] = sent_v
    sbuf[pl.ds(32, 16)] = sent_v
    vregs = []
    lranks = []
    for half in range(2):
        cnt_ref[...] = zero_v
        for j in range(4):
            v = eidx_v[pl.ds(64 * half + 16 * j, 16)]
            sbuf[pl.ds(16, 16)] = v
            rank = zero_v
            later = zero_v
            for d in range(1, 16):
                rank = rank + jnp.where(
                    _shift_load(sbuf, 16 - d) == v, one_v, zero_v)
                later = later + jnp.where(
                    _shift_load(sbuf, 16 + d) == v, one_v, zero_v)
            before = plsc.load_gather(cnt_ref, [v])
            vregs.append(v)
            lranks.append(before + rank)
            plsc.store_scatter(cnt_ref, [v], before + rank + 1,
                               mask=later == zero_v)
        stage_v[half, :] = cnt_ref[...]
    pltpu.sync_copy(stage_v, gridx_hbm.at[c].at[pl.ds(2 * s, 2)])
    plsc.subcore_barrier()

    pltpu.sync_copy(gridx_hbm.at[c], grid_v)
    k = 2 * s + c
    k_v = zero_v + k
    tot = zero_v
    pre = zero_v
    for r in range(32):
        row = grid_v[r, :]
        tot = tot + row
        pre = pre + jnp.where(zero_v + r < k_v, row, zero_v)
    sbuf[pl.ds(0, 16)] = zero_v
    csum = tot
    for d in (1, 2, 4, 8):
        sbuf[pl.ds(16, 16)] = csum
        csum = csum + _shift_load(sbuf, 16 - d)
    ends = csum
    start_ref[...] = ends - tot + pre
    endv[...] = ends

    c_is_0 = (zero_v + c) == zero_v
    for j in range(4):
        v = jnp.where(c_is_0, vregs[j], vregs[4 + j])
        lr = jnp.where(c_is_0, lranks[j], lranks[4 + j])
        pos_v[pl.ds(16 * j, 16)] = plsc.load_gather(start_ref, [v]) + lr

    base = k * _CK
    pltpu.sync_copy(x_hbm.at[pl.ds(base, _CK)], xv)
    pltpu.async_copy(xv, xs_hbm.at[pos_v], sem).wait()
    pltpu.sync_copy(pos_v, pos_hbm.at[pl.ds(base, _CK)])

    @pl.when(jnp.logical_and(c == 0, s == 0))
    def _():
        pltpu.sync_copy(endv, ends_hbm)


def _combine_body(pos_hbm, ys_hbm, out_hbm, pos_v, rows_v, sem):
    c = lax.axis_index("c")
    s = lax.axis_index("s")
    base = (2 * s + c) * _CK
    pltpu.sync_copy(pos_hbm.at[pl.ds(base, _CK)], pos_v)
    pltpu.async_copy(ys_hbm.at[pos_v], rows_v, sem).wait()
    pltpu.sync_copy(rows_v, out_hbm.at[pl.ds(base, _CK)])


def _tc_body(ends_ref, xs_ref, W1_ref, b1_ref, W2_ref, b2_ref, out_ref):
    t = pl.program_id(0)
    TB, O = out_ref.shape
    rows = xs_ref[...]
    rowid = lax.broadcasted_iota(jnp.int32, (TB, O), 0) + t * TB
    out_ref[...] = jnp.zeros((TB, O), jnp.float32)
    dn = (((1,), (1,)), ((), ()))
    for e in range(_E):
        end = ends_ref[e]
        start = jnp.int32(0) if e == 0 else ends_ref[e - 1]

        @pl.when(jnp.logical_and(end > t * TB, start < (t + 1) * TB))
        def _(e=e, start=start, end=end):
            h = lax.dot_general(rows, W1_ref[e], dn,
                                preferred_element_type=jnp.float32)
            h = jax.nn.relu(h + b1_ref[e:e + 1, :])
            y = lax.dot_general(h, W2_ref[e], dn,
                                preferred_element_type=jnp.float32)
            y = y + b2_ref[e:e + 1, :]
            mask = jnp.logical_and(rowid >= start, rowid < end)
            out_ref[...] = jnp.where(mask, y, out_ref[...])


def kernel(quant_fn, x, emb_idx, W1, b1, W2, b2):
    del quant_fn
    B, X = x.shape
    E, H, _ = W1.shape
    O = W2.shape[1]
    mesh = plsc.VectorSubcoreMesh(core_axis_name="c", subcore_axis_name="s")

    sc_params = pltpu.CompilerParams(needs_layout_passes=False)
    dispatch = pl.kernel(
        _dispatch_body,
        out_type=[
            jax.ShapeDtypeStruct((B, X), jnp.float32),
            jax.ShapeDtypeStruct((B,), jnp.int32),
            jax.ShapeDtypeStruct((E,), jnp.int32),
            jax.ShapeDtypeStruct((2, 32, E), jnp.int32),
        ],
        mesh=mesh,
        scratch_types=[
            pltpu.VMEM((128,), jnp.int32),
            pltpu.VMEM((48,), jnp.int32),
            pltpu.VMEM((2, 16), jnp.int32),
            pltpu.VMEM((32, 16), jnp.int32),
            pltpu.VMEM((16,), jnp.int32),
            pltpu.VMEM((16,), jnp.int32),
            pltpu.VMEM((_CK, X), jnp.float32),
            pltpu.VMEM((_CK,), jnp.int32),
            pltpu.VMEM((16,), jnp.int32),
            pltpu.SemaphoreType.DMA,
        ],
        compiler_params=sc_params,
    )
    xs, pos, ends, _ = dispatch(emb_idx, x)

    ys = pl.pallas_call(
        _tc_body,
        grid=(B // _TB,),
        in_specs=[
            pl.BlockSpec(memory_space=pltpu.SMEM),
            pl.BlockSpec((_TB, X), lambda i: (i, 0)),
            pl.BlockSpec((E, H, X), lambda i: (0, 0, 0)),
            pl.BlockSpec((E, H), lambda i: (0, 0)),
            pl.BlockSpec((E, O, H), lambda i: (0, 0, 0)),
            pl.BlockSpec((E, O), lambda i: (0, 0)),
        ],
        out_specs=pl.BlockSpec((_TB, O), lambda i: (i, 0)),
        out_shape=jax.ShapeDtypeStruct((B, O), jnp.float32),
    )(ends, xs, W1, b1, W2, b2)

    combine = pl.kernel(
        _combine_body,
        out_type=jax.ShapeDtypeStruct((B, O), jnp.float32),
        mesh=mesh,
        scratch_types=[
            pltpu.VMEM((_CK,), jnp.int32),
            pltpu.VMEM((_CK, O), jnp.float32),
            pltpu.SemaphoreType.DMA,
        ],
        compiler_params=sc_params,
    )
    return combine(pos, ys)


# speedup vs baseline: 1.0330x; 1.0330x over previous
import jax
import jax.numpy as jnp
from jax import lax
from jax.experimental import pallas as pl
from jax.experimental.pallas import tpu as pltpu
from jax.experimental.pallas import tpu_sc as plsc

_B = 2048
_E = 16
_CK = 64
_TB = 256


def _shift_load(sbuf, off):
    return sbuf[pl.ds(off, 16)]


def _dispatch_body(eidx_hbm, x_hbm, xs_hbm, pos_hbm, ends_hbm, gridx_hbm,
                   eidx_v, sbuf, stage_v, grid_v, cnt_ref, start_ref, xv,
                   pos_v, endv, sem):
    c = lax.axis_index("c")
    s = lax.axis_index("s")
    iota = lax.iota(jnp.int32, 16)
    zero_v = iota * 0
    one_v = zero_v + 1
    sent_v = zero_v - 1

    pltpu.sync_copy(eidx_hbm.at[pl.ds(s * 128, 128)], eidx_v)
    k = 2 * s + c
    base = k * _CK
    xcp = pltpu.async_copy(x_hbm.at[pl.ds(base, _CK)], xv, sem)
    sbuf[pl.ds(0, 16)] = sent_v
    vregs = []
    lranks = []
    for half in range(2):
        cnt_ref[...] = zero_v
        for j in range(4):
            v = eidx_v[pl.ds(64 * half + 16 * j, 16)]
            sbuf[pl.ds(16, 16)] = v
            rank = zero_v
            for d in range(1, 16):
                rank = rank + jnp.where(
                    _shift_load(sbuf, 16 - d) == v, one_v, zero_v)
            before = plsc.load_gather(cnt_ref, [v])
            vregs.append(v)
            lranks.append(before + rank)
            plsc.store_scatter(cnt_ref, [v], before + rank + 1)
        stage_v[half, :] = cnt_ref[...]
    pltpu.sync_copy(stage_v, gridx_hbm.at[c].at[pl.ds(2 * s, 2)])
    plsc.subcore_barrier()

    pltpu.sync_copy(gridx_hbm.at[c], grid_v)
    k_v = zero_v + k
    tot = zero_v
    pre = zero_v
    for r in range(32):
        row = grid_v[r, :]
        tot = tot + row
        pre = pre + jnp.where(zero_v + r < k_v, row, zero_v)
    sbuf[pl.ds(0, 16)] = zero_v
    csum = tot
    for d in (1, 2, 4, 8):
        sbuf[pl.ds(16, 16)] = csum
        csum = csum + _shift_load(sbuf, 16 - d)
    ends = csum
    start_ref[...] = ends - tot + pre
    endv[...] = ends

    c_is_0 = (zero_v + c) == zero_v
    for j in range(4):
        v = jnp.where(c_is_0, vregs[j], vregs[4 + j])
        lr = jnp.where(c_is_0, lranks[j], lranks[4 + j])
        pos_v[pl.ds(16 * j, 16)] = plsc.load_gather(start_ref, [v]) + lr

    xcp.wait()
    pltpu.async_copy(xv, xs_hbm.at[pos_v], sem).wait()
    pltpu.sync_copy(pos_v, pos_hbm.at[pl.ds(base, _CK)])

    @pl.when(jnp.logical_and(c == 0, s == 0))
    def _():
        pltpu.sync_copy(endv, ends_hbm)


def _combine_body(pos_hbm, ys_hbm, out_hbm, pos_v, rows_v, sem):
    c = lax.axis_index("c")
    s = lax.axis_index("s")
    base = (2 * s + c) * _CK
    pltpu.sync_copy(pos_hbm.at[pl.ds(base, _CK)], pos_v)
    pltpu.async_copy(ys_hbm.at[pos_v], rows_v, sem).wait()
    pltpu.sync_copy(rows_v, out_hbm.at[pl.ds(base, _CK)])


def _tc_body(ends_ref, xs_ref, W1_ref, b1_ref, W2_ref, b2_ref, out_ref):
    t = pl.program_id(0)
    TB, O = out_ref.shape
    rows = xs_ref[...]
    rowid = lax.broadcasted_iota(jnp.int32, (TB, O), 0) + t * TB
    out_ref[...] = jnp.zeros((TB, O), jnp.float32)
    dn = (((1,), (1,)), ((), ()))
    for e in range(_E):
        end = ends_ref[e]
        start = jnp.int32(0) if e == 0 else ends_ref[e - 1]

        @pl.when(jnp.logical_and(end > t * TB, start < (t + 1) * TB))
        def _(e=e, start=start, end=end):
            h = lax.dot_general(rows, W1_ref[e], dn,
                                preferred_element_type=jnp.float32)
            h = jax.nn.relu(h + b1_ref[e:e + 1, :])
            y = lax.dot_general(h, W2_ref[e], dn,
                                preferred_element_type=jnp.float32)
            y = y + b2_ref[e:e + 1, :]
            mask = jnp.logical_and(rowid >= start, rowid < end)
            out_ref[...] = jnp.where(mask, y, out_ref[...])


def kernel(quant_fn, x, emb_idx, W1, b1, W2, b2):
    del quant_fn
    B, X = x.shape
    E, H, _ = W1.shape
    O = W2.shape[1]
    mesh = plsc.VectorSubcoreMesh(core_axis_name="c", subcore_axis_name="s")

    sc_params = pltpu.CompilerParams(needs_layout_passes=False)
    dispatch = pl.kernel(
        _dispatch_body,
        out_type=[
            jax.ShapeDtypeStruct((B, X), jnp.float32),
            jax.ShapeDtypeStruct((B,), jnp.int32),
            jax.ShapeDtypeStruct((E,), jnp.int32),
            jax.ShapeDtypeStruct((2, 32, E), jnp.int32),
        ],
        mesh=mesh,
        scratch_types=[
            pltpu.VMEM((128,), jnp.int32),
            pltpu.VMEM((48,), jnp.int32),
            pltpu.VMEM((2, 16), jnp.int32),
            pltpu.VMEM((32, 16), jnp.int32),
            pltpu.VMEM((16,), jnp.int32),
            pltpu.VMEM((16,), jnp.int32),
            pltpu.VMEM((_CK, X), jnp.float32),
            pltpu.VMEM((_CK,), jnp.int32),
            pltpu.VMEM((16,), jnp.int32),
            pltpu.SemaphoreType.DMA,
        ],
        compiler_params=sc_params,
    )
    xs, pos, ends, _ = dispatch(emb_idx, x)

    ys = pl.pallas_call(
        _tc_body,
        grid=(B // _TB,),
        in_specs=[
            pl.BlockSpec(memory_space=pltpu.SMEM),
            pl.BlockSpec((_TB, X), lambda i: (i, 0)),
            pl.BlockSpec((E, H, X), lambda i: (0, 0, 0)),
            pl.BlockSpec((E, H), lambda i: (0, 0)),
            pl.BlockSpec((E, O, H), lambda i: (0, 0, 0)),
            pl.BlockSpec((E, O), lambda i: (0, 0)),
        ],
        out_specs=pl.BlockSpec((_TB, O), lambda i: (i, 0)),
        out_shape=jax.ShapeDtypeStruct((B, O), jnp.float32),
    )(ends, xs, W1, b1, W2, b2)

    combine = pl.kernel(
        _combine_body,
        out_type=jax.ShapeDtypeStruct((B, O), jnp.float32),
        mesh=mesh,
        scratch_types=[
            pltpu.VMEM((_CK,), jnp.int32),
            pltpu.VMEM((_CK, O), jnp.float32),
            pltpu.SemaphoreType.DMA,
        ],
        compiler_params=sc_params,
    )
    return combine(pos, ys)
